# baseline (speedup 1.0000x reference)
import jax
import jax.numpy as jnp
from jax.experimental import pallas as pl
from jax.experimental.pallas import tpu as pltpu

_K = 7
_PAD = (_K - 1) // 2


def _roll_left(x, s):
    n = x.shape[-1]
    s = s % n
    if s == 0:
        return x
    return jnp.concatenate([x[:, s:], x[:, :s]], axis=-1)


def _conv_gate(p_max, p_sum, w_ref, shift_ref, H, W):
    HW = H * W
    iota = jax.lax.broadcasted_iota(jnp.int32, (1, HW), 1)
    w_idx = jax.lax.rem(iota, W)
    h_idx = jax.lax.div(iota, W)

    acc = jnp.full((1, HW), shift_ref[0], dtype=jnp.float32)
    for c, p in ((0, p_max.astype(jnp.float32)), (1, p_sum)):
        rows = []
        for dj in range(-_PAD, _PAD + 1):
            q = _roll_left(p, dj)
            valid_w = ((w_idx + dj) >= 0) & ((w_idx + dj) < W)
            rows.append(jnp.where(valid_w, q, 0.0))
        for di in range(-_PAD, _PAD + 1):
            r = rows[0] * w_ref[c, di + _PAD, 0]
            for dj in range(1, _K):
                r = r + rows[dj] * w_ref[c, di + _PAD, dj]
            r = _roll_left(r, W * di)
            valid_h = ((h_idx + di) >= 0) & ((h_idx + di) < H)
            acc = acc + jnp.where(valid_h, r, 0.0)
    return jax.nn.sigmoid(acc)


def _sam_kernel_nhwc(H, W, x_ref, w_ref, shift_ref, o_ref):
    Bt, HW, C = x_ref.shape

    for b in range(Bt):
        pm_parts, ps_parts = [], []
        for k in range(HW // 128):
            xc = x_ref[b, pl.ds(k * 128, 128), :]
            xt = jnp.transpose(xc)
            xt3 = xt.reshape(C // 8, 8, 128)
            am = jnp.max(xt3, axis=0)
            asm = jnp.sum(xt3.astype(jnp.float32), axis=0)
            pm_parts.append(jnp.max(am, axis=0, keepdims=True))
            ps_parts.append(jnp.sum(asm, axis=0, keepdims=True))
        p_max = jnp.concatenate(pm_parts, axis=1)
        p_sum = jnp.concatenate(ps_parts, axis=1)

        gate = _conv_gate(p_max, p_sum, w_ref, shift_ref, H, W)

        for k in range(HW // 128):
            gk = gate[:, k * 128:(k + 1) * 128]
            bc = jnp.broadcast_to(gk, (128, 128))
            g_col = jnp.transpose(bc)
            g_full = pltpu.repeat(g_col, C // 128, axis=1)
            xc = x_ref[b, pl.ds(k * 128, 128), :]
            o_ref[b, pl.ds(k * 128, 128), :] = (
                xc * g_full.astype(xc.dtype)).astype(o_ref.dtype)


def _sam_kernel_flat(W, x_ref, w_ref, shift_ref, o_ref):
    Bt, C, HW = x_ref.shape
    H = HW // W

    if C % 32 == 0:
        ch = 32
    elif C % 16 == 0:
        ch = 16
    elif C % 8 == 0:
        ch = 8
    else:
        ch = 1
    n_chunks = C // ch

    for b in range(Bt):
        if ch >= 8:

            def pool_body(i, carry, b=b):
                am, asm = carry
                c0 = pl.multiple_of(i * ch, ch)
                blk = x_ref[b, pl.ds(c0, ch), :]
                blk3 = blk.reshape(ch // 8, 8, HW)
                am = jnp.maximum(am, jnp.max(blk3, axis=0))
                asm = asm + jnp.sum(blk3.astype(jnp.float32), axis=0)
                return am, asm

            am, asm = jax.lax.fori_loop(
                0, n_chunks, pool_body,
                (jnp.full((8, HW), -jnp.inf, dtype=x_ref.dtype),
                 jnp.zeros((8, HW), jnp.float32)),
                unroll=2)
            p_max = jnp.max(am, axis=0, keepdims=True)
            p_sum = jnp.sum(asm, axis=0, keepdims=True)
        else:
            p_max = x_ref[b, 0:1, :]
            p_sum = p_max.astype(jnp.float32)
            for c in range(1, C):
                xc = x_ref[b, c:c + 1, :]
                p_max = jnp.maximum(p_max, xc)
                p_sum = p_sum + xc.astype(jnp.float32)

        gate = _conv_gate(p_max, p_sum, w_ref, shift_ref, H, W)
        if o_ref.dtype == jnp.bfloat16:
            gate = gate.astype(jnp.bfloat16)

        if ch >= 8:

            def gate_body(i, carry, b=b, gate=gate):
                c0 = pl.multiple_of(i * ch, ch)
                xblk = x_ref[b, pl.ds(c0, ch), :]
                o_ref[b, pl.ds(c0, ch), :] = (xblk * gate).astype(o_ref.dtype)
                return carry

            jax.lax.fori_loop(0, n_chunks, gate_body, 0, unroll=2)
        else:
            o_ref[b] = (x_ref[b] * gate).astype(o_ref.dtype)


def _pick_batch_tile(B, block_bytes, target_bytes=4 * 1024 * 1024):
    bt = max(1, min(B, target_bytes // max(block_bytes, 1)))
    while bt > 1 and B // bt < 2:
        bt -= 1
    while B % bt:
        bt -= 1
    return bt


def kernel(x, conv_w, bn_gamma, bn_beta, bn_mean, bn_var, eps=1e-5):
    B, C, H, W = x.shape
    HW = H * W

    bn_scale = bn_gamma / jnp.sqrt(bn_var + eps)
    bn_shift = bn_beta - bn_mean * bn_scale

    w = conv_w.reshape(2, _K, _K).astype(jnp.float32)
    w_folded = jnp.stack([w[0] * bn_scale, w[1] * (bn_scale / C)])
    shift_arr = jnp.reshape(bn_shift, (1,)).astype(jnp.float32)

    if C % 128 == 0 and HW % 128 == 0:
        x_nhwc = jnp.transpose(x, (0, 2, 3, 1)).reshape(B, HW, C)
        bt = _pick_batch_tile(B, HW * C * x.dtype.itemsize,
                              target_bytes=8 * 1024 * 1024)

        def body(x_ref, w_ref, shift_ref, o_ref):
            return _sam_kernel_nhwc(H, W, x_ref, w_ref, shift_ref, o_ref)

        out_t = pl.pallas_call(
            body,
            out_shape=jax.ShapeDtypeStruct((B, HW, C), x.dtype),
            grid=(B // bt,),
            in_specs=[
                pl.BlockSpec((bt, HW, C), lambda i: (i, 0, 0)),
                pl.BlockSpec(memory_space=pltpu.MemorySpace.SMEM),
                pl.BlockSpec(memory_space=pltpu.MemorySpace.SMEM),
            ],
            out_specs=pl.BlockSpec((bt, HW, C), lambda i: (i, 0, 0)),
            compiler_params=pltpu.CompilerParams(
                dimension_semantics=("parallel",),
                vmem_limit_bytes=48 * 1024 * 1024,
            ),
        )(x_nhwc, w_folded, shift_arr)

        return jnp.transpose(out_t.reshape(B, H, W, C), (0, 3, 1, 2))

    x_flat = x.reshape(B, C, HW)
    bt = _pick_batch_tile(B, C * HW * x.dtype.itemsize)

    def body(x_ref, w_ref, shift_ref, o_ref):
        return _sam_kernel_flat(W, x_ref, w_ref, shift_ref, o_ref)

    out_flat = pl.pallas_call(
        body,
        out_shape=jax.ShapeDtypeStruct((B, C, HW), x.dtype),
        grid=(B // bt,),
        in_specs=[
            pl.BlockSpec((bt, C, HW), lambda i: (i, 0, 0)),
            pl.BlockSpec(memory_space=pltpu.MemorySpace.SMEM),
            pl.BlockSpec(memory_space=pltpu.MemorySpace.SMEM),
        ],
        out_specs=pl.BlockSpec((bt, C, HW), lambda i: (i, 0, 0)),
        compiler_params=pltpu.CompilerParams(
            dimension_semantics=("parallel",),
            vmem_limit_bytes=48 * 1024 * 1024,
        ),
    )(x_flat, w_folded, shift_arr)

    return out_flat.reshape(B, C, H, W)


# speedup vs baseline: 1.0580x; 1.0580x over previous
import jax
import jax.numpy as jnp
from jax.experimental import pallas as pl
from jax.experimental.pallas import tpu as pltpu

_K = 7
_PAD = (_K - 1) // 2


def _roll_left(x, s):
    n = x.shape[-1]
    s = s % n
    if s == 0:
        return x
    return jnp.concatenate([x[:, s:], x[:, :s]], axis=-1)


def _conv_gate(p_max, p_sum, w_ref, shift_ref, H, W):
    HW = H * W
    iota = jax.lax.broadcasted_iota(jnp.int32, (1, HW), 1)
    w_idx = jax.lax.rem(iota, W)
    h_idx = jax.lax.div(iota, W)

    acc = jnp.full((1, HW), shift_ref[0], dtype=jnp.float32)
    for c, p in ((0, p_max.astype(jnp.float32)), (1, p_sum)):
        rows = []
        for dj in range(-_PAD, _PAD + 1):
            q = _roll_left(p, dj)
            valid_w = ((w_idx + dj) >= 0) & ((w_idx + dj) < W)
            rows.append(jnp.where(valid_w, q, 0.0))
        for di in range(-_PAD, _PAD + 1):
            r = rows[0] * w_ref[c, di + _PAD, 0]
            for dj in range(1, _K):
                r = r + rows[dj] * w_ref[c, di + _PAD, dj]
            r = _roll_left(r, W * di)
            valid_h = ((h_idx + di) >= 0) & ((h_idx + di) < H)
            acc = acc + jnp.where(valid_h, r, 0.0)
    return jax.nn.sigmoid(acc)


def _sam_kernel_nhwc(H, W, x_ref, w_ref, shift_ref, o_ref):
    Bt, HW, C = x_ref.shape

    for b in range(Bt):
        pm_parts, ps_parts = [], []
        for k in range(HW // 128):
            xc = x_ref[b, pl.ds(k * 128, 128), :]
            xt = jnp.transpose(xc)
            xt3 = xt.reshape(C // 8, 8, 128)
            am = jnp.max(xt3, axis=0)
            asm = jnp.sum(xt3.astype(jnp.float32), axis=0)
            pm_parts.append(jnp.max(am, axis=0, keepdims=True))
            ps_parts.append(jnp.sum(asm, axis=0, keepdims=True))
        p_max = jnp.concatenate(pm_parts, axis=1)
        p_sum = jnp.concatenate(ps_parts, axis=1)

        gate = _conv_gate(p_max, p_sum, w_ref, shift_ref, H, W)

        for k in range(HW // 128):
            gk = gate[:, k * 128:(k + 1) * 128]
            bc = jnp.broadcast_to(gk, (128, 128))
            g_col = jnp.transpose(bc)
            g_full = pltpu.repeat(g_col, C // 128, axis=1)
            xc = x_ref[b, pl.ds(k * 128, 128), :]
            o_ref[b, pl.ds(k * 128, 128), :] = (
                xc * g_full.astype(xc.dtype)).astype(o_ref.dtype)


def _sam_kernel_flat(W, x_ref, w_ref, shift_ref, o_ref):
    Bt, C, HW = x_ref.shape
    H = HW // W

    if C % 32 == 0:
        ch = 32
    elif C % 16 == 0:
        ch = 16
    elif C % 8 == 0:
        ch = 8
    else:
        ch = 1
    n_chunks = C // ch

    for b in range(Bt):
        if ch >= 8:

            def pool_body(i, carry, b=b):
                am, asm = carry
                c0 = pl.multiple_of(i * ch, ch)
                blk = x_ref[b, pl.ds(c0, ch), :]
                blk3 = blk.reshape(ch // 8, 8, HW)
                am = jnp.maximum(am, jnp.max(blk3, axis=0))
                asm = asm + jnp.sum(blk3.astype(jnp.float32), axis=0)
                return am, asm

            am, asm = jax.lax.fori_loop(
                0, n_chunks, pool_body,
                (jnp.full((8, HW), -jnp.inf, dtype=x_ref.dtype),
                 jnp.zeros((8, HW), jnp.float32)),
                unroll=2)
            p_max = jnp.max(am, axis=0, keepdims=True)
            p_sum = jnp.sum(asm, axis=0, keepdims=True)
        else:
            p_max = x_ref[b, 0:1, :]
            p_sum = p_max.astype(jnp.float32)
            for c in range(1, C):
                xc = x_ref[b, c:c + 1, :]
                p_max = jnp.maximum(p_max, xc)
                p_sum = p_sum + xc.astype(jnp.float32)

        gate = _conv_gate(p_max, p_sum, w_ref, shift_ref, H, W)
        if o_ref.dtype == jnp.bfloat16:
            gate = gate.astype(jnp.bfloat16)

        if ch >= 8:

            def gate_body(i, carry, b=b, gate=gate):
                c0 = pl.multiple_of(i * ch, ch)
                xblk = x_ref[b, pl.ds(c0, ch), :]
                o_ref[b, pl.ds(c0, ch), :] = (xblk * gate).astype(o_ref.dtype)
                return carry

            jax.lax.fori_loop(0, n_chunks, gate_body, 0, unroll=2)
        else:
            o_ref[b] = (x_ref[b] * gate).astype(o_ref.dtype)


def _pick_batch_tile(B, block_bytes, target_bytes=4 * 1024 * 1024):
    bt = max(1, min(B, target_bytes // max(block_bytes, 1)))
    while bt > 1 and B // bt < 2:
        bt -= 1
    while B % bt:
        bt -= 1
    return bt


def kernel(x, conv_w, bn_gamma, bn_beta, bn_mean, bn_var, eps=1e-5):
    B, C, H, W = x.shape
    HW = H * W

    bn_scale = bn_gamma / jnp.sqrt(bn_var + eps)
    bn_shift = bn_beta - bn_mean * bn_scale

    w = conv_w.reshape(2, _K, _K).astype(jnp.float32)
    w_folded = jnp.stack([w[0] * bn_scale, w[1] * (bn_scale / C)])
    shift_arr = jnp.reshape(bn_shift, (1,)).astype(jnp.float32)

    if C % 128 == 0 and HW % 128 == 0:
        x_nhwc = jnp.transpose(x, (0, 2, 3, 1)).reshape(B, HW, C)
        bt = _pick_batch_tile(B, HW * C * x.dtype.itemsize,
                              target_bytes=4 * 1024 * 1024)

        def body(x_ref, w_ref, shift_ref, o_ref):
            return _sam_kernel_nhwc(H, W, x_ref, w_ref, shift_ref, o_ref)

        out_t = pl.pallas_call(
            body,
            out_shape=jax.ShapeDtypeStruct((B, HW, C), x.dtype),
            grid=(B // bt,),
            in_specs=[
                pl.BlockSpec((bt, HW, C), lambda i: (i, 0, 0)),
                pl.BlockSpec(memory_space=pltpu.MemorySpace.SMEM),
                pl.BlockSpec(memory_space=pltpu.MemorySpace.SMEM),
            ],
            out_specs=pl.BlockSpec((bt, HW, C), lambda i: (i, 0, 0)),
            compiler_params=pltpu.CompilerParams(
                dimension_semantics=("parallel",),
                vmem_limit_bytes=48 * 1024 * 1024,
            ),
        )(x_nhwc, w_folded, shift_arr)

        return jnp.transpose(out_t.reshape(B, H, W, C), (0, 3, 1, 2))

    x_flat = x.reshape(B, C, HW)
    bt = _pick_batch_tile(B, C * HW * x.dtype.itemsize)

    def body(x_ref, w_ref, shift_ref, o_ref):
        return _sam_kernel_flat(W, x_ref, w_ref, shift_ref, o_ref)

    out_flat = pl.pallas_call(
        body,
        out_shape=jax.ShapeDtypeStruct((B, C, HW), x.dtype),
        grid=(B // bt,),
        in_specs=[
            pl.BlockSpec((bt, C, HW), lambda i: (i, 0, 0)),
            pl.BlockSpec(memory_space=pltpu.MemorySpace.SMEM),
            pl.BlockSpec(memory_space=pltpu.MemorySpace.SMEM),
        ],
        out_specs=pl.BlockSpec((bt, C, HW), lambda i: (i, 0, 0)),
        compiler_params=pltpu.CompilerParams(
            dimension_semantics=("parallel",),
            vmem_limit_bytes=48 * 1024 * 1024,
        ),
    )(x_flat, w_folded, shift_arr)

    return out_flat.reshape(B, C, H, W)


# speedup vs baseline: 1.0718x; 1.0131x over previous
import jax
import jax.numpy as jnp
from jax.experimental import pallas as pl
from jax.experimental.pallas import tpu as pltpu

_K = 7
_PAD = (_K - 1) // 2


def _roll_left(x, s):
    n = x.shape[-1]
    s = s % n
    if s == 0:
        return x
    return jnp.concatenate([x[:, s:], x[:, :s]], axis=-1)


def _conv_gate(p_max, p_sum, w_ref, sc_ref, H, W):
    HW = H * W
    iota = jax.lax.broadcasted_iota(jnp.int32, (1, HW), 1)
    w_idx = jax.lax.rem(iota, W)
    h_idx = jax.lax.div(iota, W)

    acc = jnp.full((1, HW), sc_ref[2], dtype=jnp.float32)
    for c, p in ((0, p_max.astype(jnp.float32)), (1, p_sum)):
        scale_c = sc_ref[c]
        rows = []
        for dj in range(-_PAD, _PAD + 1):
            q = _roll_left(p, dj)
            valid_w = ((w_idx + dj) >= 0) & ((w_idx + dj) < W)
            rows.append(jnp.where(valid_w, q, 0.0))
        for di in range(-_PAD, _PAD + 1):
            r = rows[0] * (w_ref[0, c, di + _PAD, 0] * scale_c)
            for dj in range(1, _K):
                r = r + rows[dj] * (w_ref[0, c, di + _PAD, dj] * scale_c)
            r = _roll_left(r, W * di)
            valid_h = ((h_idx + di) >= 0) & ((h_idx + di) < H)
            acc = acc + jnp.where(valid_h, r, 0.0)
    return jax.nn.sigmoid(acc)


def _sam_kernel_nhwc(H, W, x_ref, w_ref, sc_ref, o_ref):
    Bt, HW, C = x_ref.shape

    for b in range(Bt):
        pm_parts, ps_parts = [], []
        for k in range(HW // 128):
            xc = x_ref[b, pl.ds(k * 128, 128), :]
            xt = jnp.transpose(xc)
            xt3 = xt.reshape(C // 8, 8, 128)
            am = jnp.max(xt3, axis=0)
            asm = jnp.sum(xt3.astype(jnp.float32), axis=0)
            pm_parts.append(jnp.max(am, axis=0, keepdims=True))
            ps_parts.append(jnp.sum(asm, axis=0, keepdims=True))
        p_max = jnp.concatenate(pm_parts, axis=1)
        p_sum = jnp.concatenate(ps_parts, axis=1)

        gate = _conv_gate(p_max, p_sum, w_ref, sc_ref, H, W)

        for k in range(HW // 128):
            gk = gate[:, k * 128:(k + 1) * 128]
            bc = jnp.broadcast_to(gk, (128, 128))
            g_col = jnp.transpose(bc)
            g_full = pltpu.repeat(g_col, C // 128, axis=1)
            xc = x_ref[b, pl.ds(k * 128, 128), :]
            o_ref[b, pl.ds(k * 128, 128), :] = (
                xc * g_full.astype(xc.dtype)).astype(o_ref.dtype)


def _sam_kernel_flat(W, x_ref, w_ref, sc_ref, o_ref):
    Bt, C, HW = x_ref.shape
    H = HW // W

    if C % 32 == 0:
        ch = 32
    elif C % 16 == 0:
        ch = 16
    elif C % 8 == 0:
        ch = 8
    else:
        ch = 1
    n_chunks = C // ch

    for b in range(Bt):
        if ch >= 8:

            def pool_body(i, carry, b=b):
                am, asm = carry
                c0 = pl.multiple_of(i * ch, ch)
                blk = x_ref[b, pl.ds(c0, ch), :]
                blk3 = blk.reshape(ch // 8, 8, HW)
                am = jnp.maximum(am, jnp.max(blk3, axis=0))
                asm = asm + jnp.sum(blk3.astype(jnp.float32), axis=0)
                return am, asm

            am, asm = jax.lax.fori_loop(
                0, n_chunks, pool_body,
                (jnp.full((8, HW), -jnp.inf, dtype=x_ref.dtype),
                 jnp.zeros((8, HW), jnp.float32)),
                unroll=2)
            p_max = jnp.max(am, axis=0, keepdims=True)
            p_sum = jnp.sum(asm, axis=0, keepdims=True)
        else:
            p_max = x_ref[b, 0:1, :]
            p_sum = p_max.astype(jnp.float32)
            for c in range(1, C):
                xc = x_ref[b, c:c + 1, :]
                p_max = jnp.maximum(p_max, xc)
                p_sum = p_sum + xc.astype(jnp.float32)

        gate = _conv_gate(p_max, p_sum, w_ref, sc_ref, H, W)
        if o_ref.dtype == jnp.bfloat16:
            gate = gate.astype(jnp.bfloat16)

        if ch >= 8:

            def gate_body(i, carry, b=b, gate=gate):
                c0 = pl.multiple_of(i * ch, ch)
                xblk = x_ref[b, pl.ds(c0, ch), :]
                o_ref[b, pl.ds(c0, ch), :] = (xblk * gate).astype(o_ref.dtype)
                return carry

            jax.lax.fori_loop(0, n_chunks, gate_body, 0, unroll=2)
        else:
            o_ref[b] = (x_ref[b] * gate).astype(o_ref.dtype)


def _pick_batch_tile(B, block_bytes, target_bytes=4 * 1024 * 1024):
    bt = max(1, min(B, target_bytes // max(block_bytes, 1)))
    while bt > 1 and B // bt < 2:
        bt -= 1
    while B % bt:
        bt -= 1
    return bt


def kernel(x, conv_w, bn_gamma, bn_beta, bn_mean, bn_var, eps=1e-5):
    B, C, H, W = x.shape
    HW = H * W

    bn_scale = bn_gamma / jnp.sqrt(bn_var + eps)
    bn_shift = bn_beta - bn_mean * bn_scale

    sc = jnp.stack([bn_scale, bn_scale / C, bn_shift]).astype(jnp.float32)
    w_raw = conv_w.astype(jnp.float32)

    if C % 128 == 0 and HW % 128 == 0:
        x_nhwc = jnp.transpose(x, (0, 2, 3, 1)).reshape(B, HW, C)
        bt = _pick_batch_tile(B, HW * C * x.dtype.itemsize,
                              target_bytes=4 * 1024 * 1024)

        def body(x_ref, w_ref, sc_ref, o_ref):
            return _sam_kernel_nhwc(H, W, x_ref, w_ref, sc_ref, o_ref)

        out_t = pl.pallas_call(
            body,
            out_shape=jax.ShapeDtypeStruct((B, HW, C), x.dtype),
            grid=(B // bt,),
            in_specs=[
                pl.BlockSpec((bt, HW, C), lambda i: (i, 0, 0)),
                pl.BlockSpec(memory_space=pltpu.MemorySpace.SMEM),
                pl.BlockSpec(memory_space=pltpu.MemorySpace.SMEM),
            ],
            out_specs=pl.BlockSpec((bt, HW, C), lambda i: (i, 0, 0)),
            compiler_params=pltpu.CompilerParams(
                dimension_semantics=("parallel",),
                vmem_limit_bytes=48 * 1024 * 1024,
            ),
        )(x_nhwc, w_raw, sc)

        return jnp.transpose(out_t.reshape(B, H, W, C), (0, 3, 1, 2))

    x_flat = x.reshape(B, C, HW)
    bt = _pick_batch_tile(B, C * HW * x.dtype.itemsize)

    def body(x_ref, w_ref, sc_ref, o_ref):
        return _sam_kernel_flat(W, x_ref, w_ref, sc_ref, o_ref)

    out_flat = pl.pallas_call(
        body,
        out_shape=jax.ShapeDtypeStruct((B, C, HW), x.dtype),
        grid=(B // bt,),
        in_specs=[
            pl.BlockSpec((bt, C, HW), lambda i: (i, 0, 0)),
            pl.BlockSpec(memory_space=pltpu.MemorySpace.SMEM),
            pl.BlockSpec(memory_space=pltpu.MemorySpace.SMEM),
        ],
        out_specs=pl.BlockSpec((bt, C, HW), lambda i: (i, 0, 0)),
        compiler_params=pltpu.CompilerParams(
            dimension_semantics=("parallel",),
            vmem_limit_bytes=48 * 1024 * 1024,
        ),
    )(x_flat, w_raw, sc)

    return out_flat.reshape(B, C, H, W)
